# baseline (speedup 1.0000x reference)
import functools

import jax
import jax.numpy as jnp
from jax import lax
from jax.experimental import pallas as pl
from jax.experimental.pallas import tpu as pltpu
from jax.experimental.pallas import tpu_sc as plsc

_NC, _NS, _L = 2, 16, 16
_NW = _NC * _NS


def _make_sc_gather_max(BN, C, K):
    PW = BN // _NW
    G = 5
    NCH = PW // G
    IPC = G * K

    mesh = plsc.VectorSubcoreMesh(core_axis_name="c", subcore_axis_name="s")

    @functools.partial(
        pl.kernel,
        mesh=mesh,
        out_type=jax.ShapeDtypeStruct((BN * C,), jnp.float32),
        scratch_types=[
            pltpu.VMEM((PW * K,), jnp.int32),
            pltpu.VMEM((IPC, C), jnp.float32),
            pltpu.VMEM((IPC, C), jnp.float32),
            pltpu.VMEM((G * C,), jnp.float32),
            pltpu.SemaphoreType.DMA,
            pltpu.SemaphoreType.DMA,
        ],
    )
    def sc_kernel(x_hbm, idx_hbm, out_hbm, idx_v, rows0, rows1, outb, sem0, sem1):
        wid = lax.axis_index("s") * _NC + lax.axis_index("c")
        node0 = wid * PW
        pltpu.sync_copy(idx_hbm.at[pl.ds(node0 * K, PW * K)], idx_v)

        def start(c, buf, sem):
            pltpu.async_copy(x_hbm.at[idx_v.at[pl.ds(c * IPC, IPC)]], buf, sem)

        def wait(buf, sem):
            pltpu.make_async_copy(
                x_hbm.at[idx_v.at[pl.ds(0, IPC)]], buf, sem
            ).wait()

        CG = C // _L

        def compute(c, buf):
            @plsc.parallel_loop(0, G)
            def node_body(g):
                base = g * K
                accs = [buf[base, pl.ds(cg * _L, _L)] for cg in range(CG)]
                for k in range(1, K):
                    for cg in range(CG):
                        accs[cg] = jnp.maximum(
                            accs[cg], buf[base + k, pl.ds(cg * _L, _L)]
                        )
                for cg in range(CG):
                    outb[pl.ds(g * C + cg * _L, _L)] = accs[cg]
            pltpu.sync_copy(outb, out_hbm.at[pl.ds((node0 + c * G) * C, G * C)])

        start(0, rows0, sem0)

        def body(i, carry):
            c0 = i * 2
            start(c0 + 1, rows1, sem1)
            wait(rows0, sem0)
            compute(c0, rows0)
            start(c0 + 2, rows0, sem0)
            wait(rows1, sem1)
            compute(c0 + 1, rows1)
            return carry

        lax.fori_loop(0, (NCH - 1) // 2, body, 0)
        wait(rows0, sem0)
        compute(NCH - 1, rows0)

    return sc_kernel


def _mm_stats_kernel(a_ref, w_ref, y_ref, stats_ref):
    a = jnp.maximum(a_ref[...], 0.0)
    y = lax.dot_general(
        a, w_ref[...], (((1,), (1,)), ((), ())),
        preferred_element_type=jnp.float32,
    )
    y_ref[...] = y

    @pl.when(pl.program_id(0) == 0)
    def _():
        stats_ref[...] = jnp.zeros_like(stats_ref)

    stats_ref[0:1, :] += jnp.sum(y, axis=0, keepdims=True)
    stats_ref[1:2, :] += jnp.sum(y * y, axis=0, keepdims=True)


def _bn_kernel(y_ref, stats_ref, gamma_ref, beta_ref, out_ref, *, inv_n):
    mean = stats_ref[0:1, :] * inv_n
    ex2 = stats_ref[1:2, :] * inv_n
    var = ex2 - mean * mean
    scale = lax.rsqrt(var + 1e-5) * gamma_ref[...]
    out_ref[...] = (y_ref[...] - mean) * scale + beta_ref[...]


def kernel(x, idx, W, gamma, beta):
    B, N, C = x.shape
    K = idx.shape[-1]
    CO = W.shape[0]
    BN = B * N

    x2 = x.reshape(BN, C)
    offs = (jnp.arange(B, dtype=jnp.int32) * N)[:, None, None]
    idxf = (idx.astype(jnp.int32) + offs).reshape(BN * K)

    agg = _make_sc_gather_max(BN, C, K)(x2, idxf).reshape(BN, C)

    BKR = 2000
    nblk = BN // BKR
    y, stats = pl.pallas_call(
        _mm_stats_kernel,
        grid=(nblk,),
        in_specs=[
            pl.BlockSpec((BKR, C), lambda i: (i, 0)),
            pl.BlockSpec((CO, C), lambda i: (0, 0)),
        ],
        out_specs=[
            pl.BlockSpec((BKR, CO), lambda i: (i, 0)),
            pl.BlockSpec((8, CO), lambda i: (0, 0)),
        ],
        out_shape=[
            jax.ShapeDtypeStruct((BN, CO), jnp.float32),
            jax.ShapeDtypeStruct((8, CO), jnp.float32),
        ],
    )(agg, W)

    out = pl.pallas_call(
        functools.partial(_bn_kernel, inv_n=1.0 / BN),
        grid=(nblk,),
        in_specs=[
            pl.BlockSpec((BKR, CO), lambda i: (i, 0)),
            pl.BlockSpec((8, CO), lambda i: (0, 0)),
            pl.BlockSpec((1, CO), lambda i: (0, 0)),
            pl.BlockSpec((1, CO), lambda i: (0, 0)),
        ],
        out_specs=pl.BlockSpec((BKR, CO), lambda i: (i, 0)),
        out_shape=jax.ShapeDtypeStruct((BN, CO), jnp.float32),
    )(y, stats, gamma.reshape(1, CO), beta.reshape(1, CO))

    return out.reshape(B, N, CO)


# speedup vs baseline: 1.5144x; 1.5144x over previous
import functools

import jax
import jax.numpy as jnp
from jax import lax
from jax.experimental import pallas as pl
from jax.experimental.pallas import tpu as pltpu
from jax.experimental.pallas import tpu_sc as plsc

_NC, _NS, _L = 2, 16, 16
_NW = _NC * _NS


def _make_sc_gather_max(BN, C, K, N):
    PW = BN // _NW
    G = 5
    NCH = PW // G
    IPC = G * K
    NPC = BN // _NC
    ST = 10
    SR = N // ST

    mesh = plsc.VectorSubcoreMesh(core_axis_name="c", subcore_axis_name="s")

    @functools.partial(
        pl.kernel,
        mesh=mesh,
        out_type=jax.ShapeDtypeStruct((BN * C,), jnp.float32),
        scratch_types=[
            pltpu.VMEM((PW * K,), jnp.int32),
            pltpu.VMEM((IPC, C), jnp.float32),
            pltpu.VMEM((IPC, C), jnp.float32),
            pltpu.VMEM((G * C,), jnp.float32),
            pltpu.VMEM_SHARED((N, C), jnp.float32),
            pltpu.SemaphoreType.DMA,
            pltpu.SemaphoreType.DMA,
        ],
    )
    def sc_kernel(x_hbm, idx_hbm, out_hbm, idx_v, rows0, rows1, outb, xs,
                  sem0, sem1):
        core = lax.axis_index("c")
        s = lax.axis_index("s")
        node0 = core * NPC + s * PW
        pltpu.sync_copy(idx_hbm.at[pl.ds(node0 * K, PW * K)], idx_v)

        @pl.when(s < ST)
        def _():
            pltpu.sync_copy(
                x_hbm.at[pl.ds(core * N + s * SR, SR)],
                xs.at[pl.ds(s * SR, SR)],
            )

        plsc.subcore_barrier()

        def start(c, buf, sem):
            pltpu.async_copy(xs.at[idx_v.at[pl.ds(c * IPC, IPC)]], buf, sem)

        def wait(buf, sem):
            pltpu.make_async_copy(
                xs.at[idx_v.at[pl.ds(0, IPC)]], buf, sem
            ).wait()

        CG = C // _L

        def compute(c, buf):
            def node_body(g, carry):
                base = g * K
                accs = [buf[base, pl.ds(cg * _L, _L)] for cg in range(CG)]
                for k in range(1, K):
                    for cg in range(CG):
                        accs[cg] = jnp.maximum(
                            accs[cg], buf[base + k, pl.ds(cg * _L, _L)]
                        )
                for cg in range(CG):
                    outb[pl.ds(g * C + cg * _L, _L)] = accs[cg]
                return carry

            lax.fori_loop(0, G, node_body, 0)
            pltpu.sync_copy(outb, out_hbm.at[pl.ds((node0 + c * G) * C, G * C)])

        start(0, rows0, sem0)

        def body(i, carry):
            c0 = i * 2
            start(c0 + 1, rows1, sem1)
            wait(rows0, sem0)
            compute(c0, rows0)
            start(c0 + 2, rows0, sem0)
            wait(rows1, sem1)
            compute(c0 + 1, rows1)
            return carry

        lax.fori_loop(0, (NCH - 1) // 2, body, 0)
        wait(rows0, sem0)
        compute(NCH - 1, rows0)

    return sc_kernel


def _mm_stats_kernel(a_ref, w_ref, y_ref, stats_ref):
    a = jnp.maximum(a_ref[...], 0.0)
    y = lax.dot_general(
        a, w_ref[...], (((1,), (1,)), ((), ())),
        preferred_element_type=jnp.float32,
    )
    y_ref[...] = y

    @pl.when(pl.program_id(0) == 0)
    def _():
        stats_ref[...] = jnp.zeros_like(stats_ref)

    stats_ref[0:1, :] += jnp.sum(y, axis=0, keepdims=True)
    stats_ref[1:2, :] += jnp.sum(y * y, axis=0, keepdims=True)


def _bn_kernel(y_ref, stats_ref, gamma_ref, beta_ref, out_ref, *, inv_n):
    mean = stats_ref[0:1, :] * inv_n
    ex2 = stats_ref[1:2, :] * inv_n
    var = ex2 - mean * mean
    scale = lax.rsqrt(var + 1e-5) * gamma_ref[...]
    out_ref[...] = (y_ref[...] - mean) * scale + beta_ref[...]


def kernel(x, idx, W, gamma, beta):
    B, N, C = x.shape
    K = idx.shape[-1]
    CO = W.shape[0]
    BN = B * N

    x2 = x.reshape(BN, C)
    idxf = idx.astype(jnp.int32).reshape(BN * K)

    agg = _make_sc_gather_max(BN, C, K, N)(x2, idxf).reshape(BN, C)

    BKR = 2000
    nblk = BN // BKR
    y, stats = pl.pallas_call(
        _mm_stats_kernel,
        grid=(nblk,),
        in_specs=[
            pl.BlockSpec((BKR, C), lambda i: (i, 0)),
            pl.BlockSpec((CO, C), lambda i: (0, 0)),
        ],
        out_specs=[
            pl.BlockSpec((BKR, CO), lambda i: (i, 0)),
            pl.BlockSpec((8, CO), lambda i: (0, 0)),
        ],
        out_shape=[
            jax.ShapeDtypeStruct((BN, CO), jnp.float32),
            jax.ShapeDtypeStruct((8, CO), jnp.float32),
        ],
    )(agg, W)

    out = pl.pallas_call(
        functools.partial(_bn_kernel, inv_n=1.0 / BN),
        grid=(nblk,),
        in_specs=[
            pl.BlockSpec((BKR, CO), lambda i: (i, 0)),
            pl.BlockSpec((8, CO), lambda i: (0, 0)),
            pl.BlockSpec((1, CO), lambda i: (0, 0)),
            pl.BlockSpec((1, CO), lambda i: (0, 0)),
        ],
        out_specs=pl.BlockSpec((BKR, CO), lambda i: (i, 0)),
        out_shape=jax.ShapeDtypeStruct((BN, CO), jnp.float32),
    )(y, stats, gamma.reshape(1, CO), beta.reshape(1, CO))

    return out.reshape(B, N, CO)


# speedup vs baseline: 1.6079x; 1.0618x over previous
import functools

import jax
import jax.numpy as jnp
from jax import lax
from jax.experimental import pallas as pl
from jax.experimental.pallas import tpu as pltpu
from jax.experimental.pallas import tpu_sc as plsc

_NC, _NS, _L = 2, 16, 16
_NW = _NC * _NS


def _make_sc_gather_max(BN, C, K, N):
    PW = BN // _NW
    G = 5
    NCH = PW // G
    IPC = G * K
    NPC = BN // _NC
    ST = 10
    SR = N // ST

    mesh = plsc.VectorSubcoreMesh(core_axis_name="c", subcore_axis_name="s")

    @functools.partial(
        pl.kernel,
        mesh=mesh,
        out_type=jax.ShapeDtypeStruct((BN * C,), jnp.float32),
        scratch_types=[
            pltpu.VMEM((PW * K,), jnp.int32),
            pltpu.VMEM((IPC, C), jnp.float32),
            pltpu.VMEM((IPC, C), jnp.float32),
            pltpu.VMEM((G * C,), jnp.float32),
            pltpu.VMEM_SHARED((N, C), jnp.float32),
            pltpu.SemaphoreType.DMA,
            pltpu.SemaphoreType.DMA,
        ],
    )
    def sc_kernel(x_hbm, idx_hbm, out_hbm, idx_v, rows0, rows1, outb, xs,
                  sem0, sem1):
        core = lax.axis_index("c")
        s = lax.axis_index("s")
        node0 = core * NPC + s * PW
        pltpu.sync_copy(idx_hbm.at[pl.ds(node0 * K, PW * K)], idx_v)

        @pl.when(s < ST)
        def _():
            pltpu.sync_copy(
                x_hbm.at[pl.ds(core * N + s * SR, SR)],
                xs.at[pl.ds(s * SR, SR)],
            )

        plsc.subcore_barrier()

        def start(c, buf, sem):
            pltpu.async_copy(xs.at[idx_v.at[pl.ds(c * IPC, IPC)]], buf, sem)

        def wait(buf, sem):
            pltpu.make_async_copy(
                xs.at[idx_v.at[pl.ds(0, IPC)]], buf, sem
            ).wait()

        CG = C // _L

        def compute(c, buf):
            def node_body(g, carry):
                base = g * K
                accs = [buf[base, pl.ds(cg * _L, _L)] for cg in range(CG)]
                for k in range(1, K):
                    for cg in range(CG):
                        accs[cg] = jnp.maximum(
                            accs[cg], buf[base + k, pl.ds(cg * _L, _L)]
                        )
                for cg in range(CG):
                    outb[pl.ds(g * C + cg * _L, _L)] = accs[cg]
                return carry

            lax.fori_loop(0, G, node_body, 0)
            pltpu.sync_copy(outb, out_hbm.at[pl.ds((node0 + c * G) * C, G * C)])

        start(0, rows0, sem0)

        def body(i, carry):
            c0 = i * 2
            start(c0 + 1, rows1, sem1)
            wait(rows0, sem0)
            compute(c0, rows0)
            start(c0 + 2, rows0, sem0)
            wait(rows1, sem1)
            compute(c0 + 1, rows1)
            return carry

        lax.fori_loop(0, (NCH - 1) // 2, body, 0)
        wait(rows0, sem0)
        compute(NCH - 1, rows0)

    return sc_kernel


def _mm_bn_kernel(a_ref, w_ref, gamma_ref, beta_ref, out_ref, y_ref, st_ref,
                  *, bkr, inv_n):
    p = pl.program_id(0)
    i = pl.program_id(1)

    @pl.when(p == 0)
    def _():
        a = jnp.maximum(a_ref[...], 0.0)
        y = lax.dot_general(
            a, w_ref[...], (((1,), (1,)), ((), ())),
            preferred_element_type=jnp.float32,
        )
        y_ref[pl.ds(i * bkr, bkr), :] = y

        @pl.when(i == 0)
        def _():
            st_ref[...] = jnp.zeros_like(st_ref)

        st_ref[0:1, :] += jnp.sum(y, axis=0, keepdims=True)
        st_ref[1:2, :] += jnp.sum(y * y, axis=0, keepdims=True)

    @pl.when(p == 1)
    def _():
        mean = st_ref[0:1, :] * inv_n
        var = st_ref[1:2, :] * inv_n - mean * mean
        scale = lax.rsqrt(var + 1e-5) * gamma_ref[...]
        yb = y_ref[pl.ds(i * bkr, bkr), :]
        out_ref[...] = (yb - mean) * scale + beta_ref[...]


def kernel(x, idx, W, gamma, beta):
    B, N, C = x.shape
    K = idx.shape[-1]
    CO = W.shape[0]
    BN = B * N

    x2 = x.reshape(BN, C)
    idxf = idx.astype(jnp.int32).reshape(BN * K)

    agg = _make_sc_gather_max(BN, C, K, N)(x2, idxf).reshape(BN, C)

    BKR = 2000
    nblk = BN // BKR
    out = pl.pallas_call(
        functools.partial(_mm_bn_kernel, bkr=BKR, inv_n=1.0 / BN),
        grid=(2, nblk),
        in_specs=[
            pl.BlockSpec((BKR, C), lambda p, i: (i * (1 - p), 0)),
            pl.BlockSpec((CO, C), lambda p, i: (0, 0)),
            pl.BlockSpec((1, CO), lambda p, i: (0, 0)),
            pl.BlockSpec((1, CO), lambda p, i: (0, 0)),
        ],
        out_specs=pl.BlockSpec((BKR, CO), lambda p, i: (i * p, 0)),
        out_shape=jax.ShapeDtypeStruct((BN, CO), jnp.float32),
        scratch_shapes=[
            pltpu.VMEM((BN, CO), jnp.float32),
            pltpu.VMEM((8, CO), jnp.float32),
        ],
    )(agg, W, gamma.reshape(1, CO), beta.reshape(1, CO))

    return out.reshape(B, N, CO)
